# baseline (speedup 1.0000x reference)
import functools

import jax
import jax.numpy as jnp
from jax import lax
from jax.experimental import pallas as pl
from jax.experimental.pallas import tpu as pltpu
from jax.experimental.pallas import tpu_sc as plsc

D = 64
NW = 32
CHUNK = 128


@jax.jit
def _embed(x_flat, table):
    B = x_flat.shape[0]
    b_per_w = B // NW
    n_chunks = b_per_w // CHUNK
    x3 = x_flat.reshape(NW, n_chunks, CHUNK)

    mesh = plsc.VectorSubcoreMesh(core_axis_name="c", subcore_axis_name="s")

    @functools.partial(
        pl.kernel,
        mesh=mesh,
        out_type=jax.ShapeDtypeStruct((B, D), jnp.float32),
        scratch_types=[
            pltpu.VMEM((n_chunks, CHUNK), jnp.int32),
            pltpu.VMEM((CHUNK, D), jnp.float32),
            pltpu.SemaphoreType.DMA,
        ],
        compiler_params=pltpu.CompilerParams(use_tc_tiling_on_sc=False),
    )
    def emb(x_hbm, table_hbm, out_hbm, idx_v, rows_v, sem):
        wid = lax.axis_index("s") * 2 + lax.axis_index("c")
        base = wid * b_per_w
        pltpu.sync_copy(x_hbm.at[wid], idx_v)

        def body(j, _):
            pltpu.async_copy(table_hbm.at[idx_v.at[j]], rows_v, sem).wait()
            pltpu.sync_copy(rows_v, out_hbm.at[pl.ds(base + j * CHUNK, CHUNK)])
            return 0

        lax.fori_loop(0, n_chunks, body, 0)

    return emb(x3, table)


def kernel(x, table):
    out = _embed(x.reshape(-1).astype(jnp.int32), table)
    return out.reshape(x.shape[0], x.shape[1], D)


# speedup vs baseline: 1.1145x; 1.1145x over previous
import functools

import jax
import jax.numpy as jnp
from jax import lax
from jax.experimental import pallas as pl
from jax.experimental.pallas import tpu as pltpu
from jax.experimental.pallas import tpu_sc as plsc

D = 64
NW = 32
CHUNK = 128
NBUF = 8
LA = 4


@jax.jit
def _embed(x_flat, table):
    B = x_flat.shape[0]
    b_per_w = B // NW
    n_chunks = b_per_w // CHUNK
    assert n_chunks % NBUF == 0 and n_chunks > NBUF
    x3 = x_flat.reshape(NW, n_chunks, CHUNK)

    mesh = plsc.VectorSubcoreMesh(core_axis_name="c", subcore_axis_name="s")

    @functools.partial(
        pl.kernel,
        mesh=mesh,
        out_type=jax.ShapeDtypeStruct((B, D), jnp.float32),
        scratch_types=[
            pltpu.VMEM((n_chunks, CHUNK), jnp.int32),
            pltpu.VMEM((NBUF, CHUNK, D), jnp.float32),
            pltpu.SemaphoreType.DMA((NBUF,)),
            pltpu.SemaphoreType.DMA((NBUF,)),
        ],
        compiler_params=pltpu.CompilerParams(use_tc_tiling_on_sc=False),
    )
    def emb(x_hbm, table_hbm, out_hbm, idx_v, rows_v, gsem, osem):
        wid = lax.axis_index("s") * 2 + lax.axis_index("c")
        base = wid * b_per_w
        pltpu.sync_copy(x_hbm.at[wid], idx_v)

        def gather(s, b):
            pltpu.async_copy(table_hbm.at[idx_v.at[s]], rows_v.at[b], gsem.at[b])

        def out_slice(s):
            return out_hbm.at[pl.ds(base + s * CHUNK, CHUNK)]

        for b in range(LA):
            gather(b, b)

        def outer(t, _):
            j0 = t * NBUF
            for b in range(NBUF):
                s = j0 + b
                pltpu.make_async_copy(
                    table_hbm.at[idx_v.at[s]], rows_v.at[b], gsem.at[b]
                ).wait()
                pltpu.async_copy(rows_v.at[b], out_slice(s), osem.at[b])
                f = s + LA
                bf = (b + LA) % NBUF

                @pl.when(jnp.logical_and(f < n_chunks, f >= NBUF))
                def _():
                    pltpu.make_async_copy(
                        rows_v.at[bf], out_slice(f - NBUF), osem.at[bf]
                    ).wait()

                @pl.when(f < n_chunks)
                def _():
                    gather(f, bf)
            return 0

        lax.fori_loop(0, n_chunks // NBUF, outer, 0)

        for b in range(NBUF):
            s = n_chunks - NBUF + b
            pltpu.make_async_copy(rows_v.at[b], out_slice(s), osem.at[b]).wait()

    return emb(x3, table)


def kernel(x, table):
    out = _embed(x.reshape(-1).astype(jnp.int32), table)
    return out.reshape(x.shape[0], x.shape[1], D)
